# baseline (speedup 1.0000x reference)
import jax
import jax.numpy as jnp
from jax import lax
from jax.experimental import pallas as pl
from jax.experimental.pallas import tpu as pltpu
from jax.experimental.pallas import tpu_sc as plsc

_DIM = 64
_SCALE = 8.0
_CHUNK = 128
_NW = 32
_NBUF = 4


def _sc_embed(x2d, table):
    n_chunks = x2d.shape[0]
    n_rows = n_chunks * _CHUNK
    chunks_per_w = n_chunks // _NW
    ngroups = chunks_per_w // _NBUF

    mesh = plsc.VectorSubcoreMesh(core_axis_name="c", subcore_axis_name="s")

    @pl.kernel(
        out_type=jax.ShapeDtypeStruct((n_rows, _DIM), jnp.float32),
        mesh=mesh,
        scratch_types=[
            pltpu.VMEM((chunks_per_w, _CHUNK), jnp.int32),
            [pltpu.VMEM((_CHUNK, _DIM), jnp.float32) for _ in range(_NBUF)],
            [pltpu.VMEM((_CHUNK, _DIM), jnp.float32) for _ in range(_NBUF)],
            [pltpu.SemaphoreType.DMA for _ in range(_NBUF)],
            [pltpu.SemaphoreType.DMA for _ in range(_NBUF)],
        ],
        compiler_params=pltpu.CompilerParams(use_tc_tiling_on_sc=False),
    )
    def k(x_hbm, table_hbm, out_hbm, idx_v, gbufs, wbufs, gsems, wsems):
        wid = lax.axis_index("s") * 2 + lax.axis_index("c")
        crow0 = wid * chunks_per_w
        pltpu.sync_copy(x_hbm.at[pl.ds(crow0, chunks_per_w)], idx_v)

        def start_gather(j, b):
            pltpu.async_copy(table_hbm.at[idx_v.at[j]], gbufs[b], gsems[b])

        def wait_gather(b):
            pltpu.make_async_copy(
                table_hbm.at[idx_v.at[0]], gbufs[b], gsems[b]
            ).wait()

        def start_write(j, b):
            pltpu.async_copy(
                wbufs[b],
                out_hbm.at[pl.ds((crow0 + j) * _CHUNK, _CHUNK)],
                wsems[b],
            )

        def wait_write(b):
            pltpu.make_async_copy(
                wbufs[b], out_hbm.at[pl.ds(0, _CHUNK)], wsems[b]
            ).wait()

        def scale(b):
            def srow(r, c):
                for kk in range(_DIM // 16):
                    sl = pl.ds(kk * 16, 16)
                    wbufs[b][r, sl] = gbufs[b][r, sl] * _SCALE
                return c

            lax.fori_loop(0, _CHUNK, srow, 0, unroll=4)

        for b in range(_NBUF):
            start_gather(b, b)

        for b in range(_NBUF):
            wait_gather(b)
            scale(b)
            start_gather(_NBUF + b, b)
            start_write(b, b)

        def group(g, c):
            for b in range(_NBUF):
                j = g * _NBUF + b
                wait_write(b)
                wait_gather(b)
                scale(b)
                start_gather(j + _NBUF, b)
                start_write(j, b)
            return c

        lax.fori_loop(1, ngroups - 1, group, 0)

        for b in range(_NBUF):
            j = (ngroups - 1) * _NBUF + b
            wait_write(b)
            wait_gather(b)
            scale(b)
            start_write(j, b)

        for b in range(_NBUF):
            wait_write(b)

    return k(x2d, table)


def kernel(x, table):
    b, s = x.shape
    x2d = x.reshape(-1, _CHUNK).astype(jnp.int32)
    out = _sc_embed(x2d, table)
    return out.reshape(b, s, _DIM)


# speedup vs baseline: 1.1145x; 1.1145x over previous
import jax
import jax.numpy as jnp
from jax import lax
from jax.experimental import pallas as pl
from jax.experimental.pallas import tpu as pltpu
from jax.experimental.pallas import tpu_sc as plsc

_DIM = 64
_SCALE = 8.0
_NW = 32


def _sc_embed(x, table):
    n, s = x.shape
    rows_per_w = n // _NW

    mesh = plsc.VectorSubcoreMesh(core_axis_name="c", subcore_axis_name="s")

    @pl.kernel(
        out_type=jax.ShapeDtypeStruct((n, s, _DIM), jnp.float32),
        mesh=mesh,
        scratch_types=[
            pltpu.VMEM((rows_per_w, s), jnp.int32),
            [pltpu.VMEM((s, _DIM), jnp.float32) for _ in range(2)],
            [pltpu.SemaphoreType.DMA for _ in range(2)],
        ],
        compiler_params=pltpu.CompilerParams(use_tc_tiling_on_sc=False),
    )
    def k(x_hbm, table_hbm, out_hbm, idx_v, bufs, sems):
        wid = lax.axis_index("s") * 2 + lax.axis_index("c")
        i0 = wid * rows_per_w
        pltpu.sync_copy(x_hbm.at[pl.ds(i0, rows_per_w)], idx_v)

        def start_gather(j, b):
            pltpu.async_copy(table_hbm.at[idx_v.at[j]], bufs[b], sems[b])

        def wait_gather(b):
            pltpu.make_async_copy(
                table_hbm.at[idx_v.at[0]], bufs[b], sems[b]
            ).wait()

        def process(j, b):
            wait_gather(b)

            def srow(r, c):
                for kk in range(_DIM // 16):
                    sl = pl.ds(kk * 16, 16)
                    bufs[b][r, sl] = bufs[b][r, sl] * _SCALE
                return c

            lax.fori_loop(0, s, srow, 0, unroll=2)
            pltpu.sync_copy(bufs[b], out_hbm.at[i0 + j])

        start_gather(0, 0)

        def pair(g, c):
            j = g * 2
            start_gather(j + 1, 1)
            process(j, 0)
            start_gather(j + 2, 0)
            process(j + 1, 1)
            return c

        lax.fori_loop(0, rows_per_w // 2 - 1, pair, 0)

        j_last = rows_per_w - 2
        start_gather(j_last + 1, 1)
        process(j_last, 0)
        process(j_last + 1, 1)

    return k(x, table)


def kernel(x, table):
    return _sc_embed(x.astype(jnp.int32), table)
